# baseline (speedup 1.0000x reference)
import functools

import jax
import jax.numpy as jnp
from jax import lax
from jax.experimental import pallas as pl
from jax.experimental.pallas import tpu as pltpu
from jax.experimental.pallas import tpu_sc as plsc

NB_CLASS = 100000
DIM = 128
BATCH = 16384
LOSS_WEIGHT = 0.01
ALPHA = 0.05
EPS = 1e-6

NC = 2
NS = 16
NW = NC * NS
BW = BATCH // NW
BS = BATCH // NS
HALF = BATCH // 2
ROWS_PER_TILE = NB_CLASS // NS

_mesh = plsc.VectorSubcoreMesh(core_axis_name="c", subcore_axis_name="s")
_f32 = jnp.float32
_i32 = jnp.int32


@functools.partial(
    pl.kernel,
    mesh=_mesh,
    out_type=(
        jax.ShapeDtypeStruct((BATCH, DIM), _f32),
        jax.ShapeDtypeStruct((128, 128), _i32),
        jax.ShapeDtypeStruct((128, 128), _f32),
    ),
    scratch_types=[
        pltpu.VMEM((4, 128), _i32),
        pltpu.VMEM((BW, DIM), _f32),
        pltpu.VMEM((8, 128), _i32),
        pltpu.VMEM((8, 128), _i32),
        pltpu.VMEM((128,), _f32),
        pltpu.VMEM((128,), _f32),
        pltpu.VMEM((8, 128), _f32),
        pltpu.VMEM_SHARED((NB_CLASS,), _i32),
        pltpu.VMEM_SHARED((NB_CLASS,), _f32),
        pltpu.SemaphoreType.DMA,
        pltpu.SemaphoreType.DMA,
    ],
)
def _k1(centers_hbm, y_hbm, sids_hbm, ones_hbm, zeros_hbm,
        cb_hbm, slots_hbm, appear_hbm,
        yg_v, rows_v, ys_v, sid_v, ones_v, zeros_v, app_v, f_sh, cnt_sh,
        sem_g, sem_p):
    c = lax.axis_index("c")
    s = lax.axis_index("s")
    wid = s * NC + c
    gbase = wid * BW
    sbase = s * BS

    for j in range(4):
        pltpu.sync_copy(y_hbm.at[pl.ds(gbase + j * 128, 128)], yg_v.at[j])
    gds = [pltpu.async_copy(centers_hbm.at[yg_v.at[j]],
                            rows_v.at[pl.ds(j * 128, 128)], sem_g)
           for j in range(4)]
    for j in range(8):
        pltpu.sync_copy(y_hbm.at[pl.ds(sbase + j * 128, 128)], ys_v.at[j])

    @pl.when(c == 0)
    def _():
        for j in range(8):
            pltpu.sync_copy(sids_hbm.at[pl.ds(sbase + j * 128, 128)], sid_v.at[j])
        ds_ = [pltpu.async_copy(sid_v.at[j], f_sh.at[ys_v.at[j]], sem_p)
               for j in range(8)]
        for d in ds_:
            d.wait()

    @pl.when(c == 1)
    def _():
        pltpu.sync_copy(ones_hbm, ones_v)
        pltpu.sync_copy(zeros_hbm, zeros_v)
        ds_ = [pltpu.async_copy(zeros_v, cnt_sh.at[ys_v.at[j]], sem_p)
               for j in range(8)]
        for d in ds_:
            d.wait()

    plsc.subcore_barrier()

    @pl.when(c == 1)
    def _():
        ds_ = [pltpu.async_copy(ones_v, cnt_sh.at[ys_v.at[j]], sem_p, add=True)
               for j in range(8)]
        for d in ds_:
            d.wait()

    plsc.subcore_barrier()

    @pl.when(c == 0)
    def _():
        ds_ = [pltpu.async_copy(f_sh.at[ys_v.at[j]], sid_v.at[j], sem_p)
               for j in range(8)]
        for d in ds_:
            d.wait()
        pltpu.sync_copy(sid_v, slots_hbm.at[pl.ds(s * 8, 8)])

    @pl.when(c == 1)
    def _():
        ds_ = [pltpu.async_copy(cnt_sh.at[ys_v.at[j]], app_v.at[j], sem_p)
               for j in range(8)]
        for d in ds_:
            d.wait()
        pltpu.sync_copy(app_v, appear_hbm.at[pl.ds(s * 8, 8)])

    for d in gds:
        d.wait()
    pltpu.sync_copy(rows_v, cb_hbm.at[pl.ds(gbase, BW)])


_NBLK = 8
_BB = BATCH // _NBLK
_CGRID = 20
_CROWS = NB_CLASS // _CGRID


def _bclamp(i):
    return (jnp.minimum(i, _NBLK - 1), 0)


def _k2_body(x_ref, cb_ref, ap_ref, c_ref, u_ref, loss_ref, nc_ref):
    i = pl.program_id(0)
    nc_ref[...] = c_ref[...]

    @pl.when(i < _NBLK)
    def _():
        d = x_ref[...] - cb_ref[...]
        u_ref[...] = ALPHA * d / (ap_ref[...] + EPS)
        part = jnp.sum(d * d)

        @pl.when(i == 0)
        def _():
            loss_ref[0, 0] = 0.0

        loss_ref[0, 0] += part * (LOSS_WEIGHT / (BATCH * DIM))


_k2 = pl.pallas_call(
    _k2_body,
    grid=(_CGRID,),
    in_specs=[
        pl.BlockSpec((_BB, DIM), _bclamp),
        pl.BlockSpec((_BB, DIM), _bclamp),
        pl.BlockSpec((_BB, 1), _bclamp),
        pl.BlockSpec((_CROWS, DIM), lambda i: (i, 0)),
    ],
    out_specs=[
        pl.BlockSpec((_BB, DIM), _bclamp),
        pl.BlockSpec((1, 1), lambda i: (0, 0), memory_space=pltpu.SMEM),
        pl.BlockSpec((_CROWS, DIM), lambda i: (i, 0)),
    ],
    out_shape=[
        jax.ShapeDtypeStruct((BATCH, DIM), _f32),
        jax.ShapeDtypeStruct((1, 1), _f32),
        jax.ShapeDtypeStruct((NB_CLASS, DIM), _f32),
    ],
)


@functools.partial(
    pl.kernel,
    mesh=_mesh,
    out_type=jax.ShapeDtypeStruct((BATCH, DIM), _f32),
    scratch_types=[
        pltpu.VMEM((8, 128), _i32),
        pltpu.VMEM((8, 128), _i32),
        pltpu.VMEM((128, DIM), _f32),
        pltpu.VMEM((2, 128, DIM), _f32),
        pltpu.VMEM_SHARED((HALF + 128, DIM), _f32),
        pltpu.SemaphoreType.DMA,
        pltpu.SemaphoreType.DMA,
        pltpu.SemaphoreType.DMA,
    ],
)
def _k3(u_hbm, slots_hbm, zrows_hbm, a_hbm,
        sl_v, li_v, z_v, u2_v, a_sh, sem_z, sem_u, sem_a):
    c = lax.axis_index("c")
    s = lax.axis_index("s")
    sbase = s * BS
    lo = c * HALF

    zd = pltpu.async_copy(zrows_hbm, z_v, sem_z)
    for j in range(8):
        pltpu.sync_copy(slots_hbm.at[pl.ds(sbase + j * 128, 128)], sl_v.at[j])

    dummy = HALF + lax.iota(_i32, 16) * 8
    @pl.loop(0, 8)
    def _(j):
        @pl.loop(0, 8)
        def _(k):
            sl = sl_v[j, pl.ds(k * 16, 16)]
            l = sl - lo
            ok = (l >= 0) & (l < HALF)
            li_v[j, pl.ds(k * 16, 16)] = jnp.where(ok, l, dummy)

    zd.wait()
    uds = [pltpu.async_copy(u_hbm.at[pl.ds(sbase + j * 128, 128)],
                            u2_v.at[j % 2], sem_u) for j in range(2)]
    zds = [pltpu.async_copy(z_v, a_sh.at[pl.ds(s * 512 + j * 128, 128)], sem_z)
           for j in range(4)]
    for d in zds:
        d.wait()
    plsc.subcore_barrier()

    ads = []
    for j in range(8):
        uds[j].wait()
        ads.append(pltpu.async_copy(u2_v.at[j % 2], a_sh.at[li_v.at[j]],
                                    sem_a, add=True))
        if j + 2 < 8:
            ads[j].wait()
            uds.append(pltpu.async_copy(
                u_hbm.at[pl.ds(sbase + (j + 2) * 128, 128)],
                u2_v.at[j % 2], sem_u))
    ads[6].wait()
    ads[7].wait()
    plsc.subcore_barrier()

    pltpu.sync_copy(a_sh.at[pl.ds(s * (HALF // NS), HALF // NS)],
                    a_hbm.at[pl.ds(lo + s * (HALF // NS), HALF // NS)])


_CPROWS = 4000


def _copy_body(src_ref, dst_ref):
    dst_ref[...] = src_ref[...]


_tc_copy = pl.pallas_call(
    _copy_body,
    grid=(NB_CLASS // _CPROWS,),
    in_specs=[pl.BlockSpec((_CPROWS, DIM), lambda i: (i, 0))],
    out_specs=pl.BlockSpec((_CPROWS, DIM), lambda i: (i, 0)),
    out_shape=jax.ShapeDtypeStruct((NB_CLASS, DIM), _f32),
)


def _scatter_rows(newc, cb, a, slots, y):
    def body(refs):
        out_ref, cb_ref, a_ref, slots_ref, y_ref = refs

        @pl.core_map(
            _mesh,
            scratch_shapes=[
                pltpu.VMEM((4, 128), _i32),
                pltpu.VMEM((4, 128), _i32),
                pltpu.VMEM((2, 128, DIM), _f32),
                pltpu.VMEM((2, 128, DIM), _f32),
                pltpu.SemaphoreType.DMA,
                pltpu.SemaphoreType.DMA,
            ],
        )
        def _(sl_v, y_v, cbr_v, ar_v, sem_i, sem_o):
            c = lax.axis_index("c")
            s = lax.axis_index("s")
            base = (s * NC + c) * BW
            for j in range(4):
                pltpu.sync_copy(slots_ref.at[pl.ds(base + j * 128, 128)], sl_v.at[j])
                pltpu.sync_copy(y_ref.at[pl.ds(base + j * 128, 128)], y_v.at[j])

            def fire(j):
                return (
                    pltpu.async_copy(cb_ref.at[pl.ds(base + j * 128, 128)],
                                     cbr_v.at[j % 2], sem_i),
                    pltpu.async_copy(a_ref.at[sl_v.at[j]], ar_v.at[j % 2], sem_i),
                )

            loads = [fire(0), fire(1)]
            outs = []
            for j in range(4):
                for d in loads[j]:
                    d.wait()

                @pl.loop(0, 128)
                def _(r):
                    for k in range(8):
                        ar_v[j % 2, r, pl.ds(k * 16, 16)] += (
                            cbr_v[j % 2, r, pl.ds(k * 16, 16)])

                outs.append(pltpu.async_copy(ar_v.at[j % 2],
                                             out_ref.at[y_v.at[j]], sem_o))
                if j + 2 < 4:
                    outs[j].wait()
                    loads.append(fire(j + 2))
            outs[2].wait()
            outs[3].wait()

    out, _, _, _, _ = pl.run_state(body)((newc, cb, a, slots, y))
    return out


def kernel(x, y, centers):
    sids = jnp.arange(BATCH, dtype=_i32)
    ones = jnp.ones((128,), _f32)
    zeros = jnp.zeros((128,), _f32)
    zrows = jnp.zeros((128, DIM), _f32)

    c_batch, slots2, appear2 = _k1(centers, y, sids, ones, zeros)
    slots = slots2.reshape(BATCH)
    u, loss, newc0 = _k2(x, c_batch, appear2.reshape(BATCH, 1), centers)
    a = _k3(u, slots, zrows)
    new_centers = _scatter_rows(newc0, c_batch, a, slots, y)
    return (loss[0, 0], new_centers)


# speedup vs baseline: 1.1067x; 1.1067x over previous
import functools

import jax
import jax.numpy as jnp
from jax import lax
from jax.experimental import pallas as pl
from jax.experimental.pallas import tpu as pltpu
from jax.experimental.pallas import tpu_sc as plsc

NB_CLASS = 100000
DIM = 128
BATCH = 16384
LOSS_WEIGHT = 0.01
ALPHA = 0.05
EPS = 1e-6

NC = 2
NS = 16
NW = NC * NS
BW = BATCH // NW
BS = BATCH // NS
HALF = BATCH // 2
ROWS_PER_TILE = NB_CLASS // NS

_mesh = plsc.VectorSubcoreMesh(core_axis_name="c", subcore_axis_name="s")
_f32 = jnp.float32
_i32 = jnp.int32


@functools.partial(
    pl.kernel,
    mesh=_mesh,
    out_type=(
        jax.ShapeDtypeStruct((BATCH, DIM), _f32),
        jax.ShapeDtypeStruct((128, 128), _i32),
        jax.ShapeDtypeStruct((128, 128), _f32),
    ),
    scratch_types=[
        pltpu.VMEM((4, 128), _i32),
        pltpu.VMEM((BW, DIM), _f32),
        pltpu.VMEM((8, 128), _i32),
        pltpu.VMEM((8, 128), _i32),
        pltpu.VMEM((128,), _f32),
        pltpu.VMEM((128,), _f32),
        pltpu.VMEM((8, 128), _f32),
        pltpu.VMEM_SHARED((NB_CLASS,), _i32),
        pltpu.VMEM_SHARED((NB_CLASS,), _f32),
        pltpu.SemaphoreType.DMA,
        pltpu.SemaphoreType.DMA,
    ],
)
def _k1(centers_hbm, y_hbm, sids_hbm, ones_hbm, zeros_hbm,
        cb_hbm, slots_hbm, appear_hbm,
        yg_v, rows_v, ys_v, sid_v, ones_v, zeros_v, app_v, f_sh, cnt_sh,
        sem_g, sem_p):
    c = lax.axis_index("c")
    s = lax.axis_index("s")
    wid = s * NC + c
    gbase = wid * BW
    sbase = s * BS

    for j in range(4):
        pltpu.sync_copy(y_hbm.at[pl.ds(gbase + j * 128, 128)], yg_v.at[j])
    gds = [pltpu.async_copy(centers_hbm.at[yg_v.at[j]],
                            rows_v.at[pl.ds(j * 128, 128)], sem_g)
           for j in range(4)]
    for j in range(8):
        pltpu.sync_copy(y_hbm.at[pl.ds(sbase + j * 128, 128)], ys_v.at[j])

    @pl.when(c == 0)
    def _():
        for j in range(8):
            pltpu.sync_copy(sids_hbm.at[pl.ds(sbase + j * 128, 128)], sid_v.at[j])
        ds_ = [pltpu.async_copy(sid_v.at[j], f_sh.at[ys_v.at[j]], sem_p)
               for j in range(8)]
        for d in ds_:
            d.wait()

    @pl.when(c == 1)
    def _():
        pltpu.sync_copy(ones_hbm, ones_v)
        pltpu.sync_copy(zeros_hbm, zeros_v)
        ds_ = [pltpu.async_copy(zeros_v, cnt_sh.at[ys_v.at[j]], sem_p)
               for j in range(8)]
        for d in ds_:
            d.wait()

    plsc.subcore_barrier()

    @pl.when(c == 1)
    def _():
        ds_ = [pltpu.async_copy(ones_v, cnt_sh.at[ys_v.at[j]], sem_p, add=True)
               for j in range(8)]
        for d in ds_:
            d.wait()

    plsc.subcore_barrier()

    @pl.when(c == 0)
    def _():
        ds_ = [pltpu.async_copy(f_sh.at[ys_v.at[j]], sid_v.at[j], sem_p)
               for j in range(8)]
        for d in ds_:
            d.wait()
        pltpu.sync_copy(sid_v, slots_hbm.at[pl.ds(s * 8, 8)])

    @pl.when(c == 1)
    def _():
        ds_ = [pltpu.async_copy(cnt_sh.at[ys_v.at[j]], app_v.at[j], sem_p)
               for j in range(8)]
        for d in ds_:
            d.wait()
        pltpu.sync_copy(app_v, appear_hbm.at[pl.ds(s * 8, 8)])

    for d in gds:
        d.wait()
    pltpu.sync_copy(rows_v, cb_hbm.at[pl.ds(gbase, BW)])


_NBLK = 8
_BB = BATCH // _NBLK


def _k2_body(x_ref, cb_ref, ap_ref, u_ref, loss_ref):
    i = pl.program_id(0)
    d = x_ref[...] - cb_ref[...]
    u_ref[...] = ALPHA * d / (ap_ref[...] + EPS)
    part = jnp.sum(d * d)

    @pl.when(i == 0)
    def _():
        loss_ref[0, 0] = 0.0

    loss_ref[0, 0] += part * (LOSS_WEIGHT / (BATCH * DIM))


_k2 = pl.pallas_call(
    _k2_body,
    grid=(_NBLK,),
    in_specs=[
        pl.BlockSpec((_BB, DIM), lambda i: (i, 0)),
        pl.BlockSpec((_BB, DIM), lambda i: (i, 0)),
        pl.BlockSpec((_BB, 1), lambda i: (i, 0)),
    ],
    out_specs=[
        pl.BlockSpec((_BB, DIM), lambda i: (i, 0)),
        pl.BlockSpec((1, 1), lambda i: (0, 0), memory_space=pltpu.SMEM),
    ],
    out_shape=[
        jax.ShapeDtypeStruct((BATCH, DIM), _f32),
        jax.ShapeDtypeStruct((1, 1), _f32),
    ],
)


@functools.partial(
    pl.kernel,
    mesh=_mesh,
    out_type=jax.ShapeDtypeStruct((BATCH, DIM), _f32),
    scratch_types=[
        pltpu.VMEM((8, 128), _i32),
        pltpu.VMEM((8, 128), _i32),
        pltpu.VMEM((128, DIM), _f32),
        pltpu.VMEM((2, 128, DIM), _f32),
        pltpu.VMEM_SHARED((HALF + 128, DIM), _f32),
        pltpu.SemaphoreType.DMA,
        pltpu.SemaphoreType.DMA,
        pltpu.SemaphoreType.DMA,
    ],
)
def _k3(u_hbm, slots_hbm, zrows_hbm, a_hbm,
        sl_v, li_v, z_v, u2_v, a_sh, sem_z, sem_u, sem_a):
    c = lax.axis_index("c")
    s = lax.axis_index("s")
    sbase = s * BS
    lo = c * HALF

    zd = pltpu.async_copy(zrows_hbm, z_v, sem_z)
    for j in range(8):
        pltpu.sync_copy(slots_hbm.at[pl.ds(sbase + j * 128, 128)], sl_v.at[j])

    dummy = HALF + lax.iota(_i32, 16) * 8
    @pl.loop(0, 8)
    def _(j):
        @pl.loop(0, 8)
        def _(k):
            sl = sl_v[j, pl.ds(k * 16, 16)]
            l = sl - lo
            ok = (l >= 0) & (l < HALF)
            li_v[j, pl.ds(k * 16, 16)] = jnp.where(ok, l, dummy)

    zd.wait()
    uds = [pltpu.async_copy(u_hbm.at[pl.ds(sbase + j * 128, 128)],
                            u2_v.at[j % 2], sem_u) for j in range(2)]
    zds = [pltpu.async_copy(z_v, a_sh.at[pl.ds(s * 512 + j * 128, 128)], sem_z)
           for j in range(4)]
    for d in zds:
        d.wait()
    plsc.subcore_barrier()

    ads = []
    for j in range(8):
        uds[j].wait()
        ads.append(pltpu.async_copy(u2_v.at[j % 2], a_sh.at[li_v.at[j]],
                                    sem_a, add=True))
        if j + 2 < 8:
            ads[j].wait()
            uds.append(pltpu.async_copy(
                u_hbm.at[pl.ds(sbase + (j + 2) * 128, 128)],
                u2_v.at[j % 2], sem_u))
    ads[6].wait()
    ads[7].wait()
    plsc.subcore_barrier()

    pltpu.sync_copy(a_sh.at[pl.ds(s * (HALF // NS), HALF // NS)],
                    a_hbm.at[pl.ds(lo + s * (HALF // NS), HALF // NS)])


_CPROWS = 4000


def _copy_body(src_ref, dst_ref):
    dst_ref[...] = src_ref[...]


_tc_copy = pl.pallas_call(
    _copy_body,
    grid=(NB_CLASS // _CPROWS,),
    in_specs=[pl.BlockSpec((_CPROWS, DIM), lambda i: (i, 0))],
    out_specs=pl.BlockSpec((_CPROWS, DIM), lambda i: (i, 0)),
    out_shape=jax.ShapeDtypeStruct((NB_CLASS, DIM), _f32),
)


def _scatter_rows(newc, cb, a, slots, y):
    def body(refs):
        out_ref, cb_ref, a_ref, slots_ref, y_ref = refs

        @pl.core_map(
            _mesh,
            scratch_shapes=[
                pltpu.VMEM((4, 128), _i32),
                pltpu.VMEM((4, 128), _i32),
                pltpu.VMEM((2, 128, DIM), _f32),
                pltpu.VMEM((2, 128, DIM), _f32),
                pltpu.SemaphoreType.DMA,
                pltpu.SemaphoreType.DMA,
            ],
        )
        def _(sl_v, y_v, cbr_v, ar_v, sem_i, sem_o):
            c = lax.axis_index("c")
            s = lax.axis_index("s")
            base = (s * NC + c) * BW
            for j in range(4):
                pltpu.sync_copy(slots_ref.at[pl.ds(base + j * 128, 128)], sl_v.at[j])
                pltpu.sync_copy(y_ref.at[pl.ds(base + j * 128, 128)], y_v.at[j])

            def fire(j):
                return (
                    pltpu.async_copy(cb_ref.at[pl.ds(base + j * 128, 128)],
                                     cbr_v.at[j % 2], sem_i),
                    pltpu.async_copy(a_ref.at[sl_v.at[j]], ar_v.at[j % 2], sem_i),
                )

            loads = [fire(0), fire(1)]
            outs = []
            for j in range(4):
                for d in loads[j]:
                    d.wait()

                @pl.loop(0, 128)
                def _(r):
                    for k in range(8):
                        ar_v[j % 2, r, pl.ds(k * 16, 16)] += (
                            cbr_v[j % 2, r, pl.ds(k * 16, 16)])

                outs.append(pltpu.async_copy(ar_v.at[j % 2],
                                             out_ref.at[y_v.at[j]], sem_o))
                if j + 2 < 4:
                    outs[j].wait()
                    loads.append(fire(j + 2))
            outs[2].wait()
            outs[3].wait()

    out, _, _, _, _ = pl.run_state(body)((newc, cb, a, slots, y))
    return out


def kernel(x, y, centers):
    sids = jnp.arange(BATCH, dtype=_i32)
    ones = jnp.ones((128,), _f32)
    zeros = jnp.zeros((128,), _f32)
    zrows = jnp.zeros((128, DIM), _f32)

    newc0 = _tc_copy(centers)
    c_batch, slots2, appear2 = _k1(centers, y, sids, ones, zeros)
    slots = slots2.reshape(BATCH)
    u, loss = _k2(x, c_batch, appear2.reshape(BATCH, 1))
    a = _k3(u, slots, zrows)
    new_centers = _scatter_rows(newc0, c_batch, a, slots, y)
    return (loss[0, 0], new_centers)
